# baseline (speedup 1.0000x reference)
import functools

import jax
import jax.numpy as jnp
from jax import lax
from jax.experimental import pallas as pl
from jax.experimental.pallas import tpu as pltpu
from jax.experimental.pallas import tpu_sc as plsc

NC = 2
NS = 16
NW = NC * NS
CHUNK = 128


@functools.partial(jax.jit, static_argnames=("n_chunks", "d"))
def _lookup(emb_table, idx3, *, n_chunks, d):
    b_total = NW * n_chunks * CHUNK
    mesh = plsc.VectorSubcoreMesh(core_axis_name="c", subcore_axis_name="s")

    def body(table_hbm, idx_hbm, out_hbm, idx_v, rows_v, gsem):
        wid = lax.axis_index("s") * NC + lax.axis_index("c")
        pltpu.sync_copy(idx_hbm.at[wid], idx_v)
        base = wid * (n_chunks * CHUNK)

        def step(j, carry):
            pltpu.async_copy(table_hbm.at[idx_v.at[j]], rows_v, gsem).wait()
            pltpu.sync_copy(rows_v, out_hbm.at[pl.ds(base + j * CHUNK, CHUNK)])
            return carry

        lax.fori_loop(0, n_chunks, step, 0)

    k = pl.kernel(
        body,
        out_type=jax.ShapeDtypeStruct((b_total, d), jnp.float32),
        mesh=mesh,
        compiler_params=pltpu.CompilerParams(use_tc_tiling_on_sc=False),
        scratch_types=[
            pltpu.VMEM((n_chunks, CHUNK), jnp.int32),
            pltpu.VMEM((CHUNK, d), jnp.float32),
            pltpu.SemaphoreType.DMA,
        ],
    )
    return k(emb_table, idx3)


def kernel(x, emb_table):
    b_total = x.size
    d = emb_table.shape[1]
    assert b_total % (NW * CHUNK) == 0
    n_chunks = b_total // (NW * CHUNK)
    idx3 = x.reshape(NW, n_chunks, CHUNK).astype(jnp.int32)
    out = _lookup(emb_table, idx3, n_chunks=n_chunks, d=d)
    return out.reshape(x.shape + (d,))


# speedup vs baseline: 1.1154x; 1.1154x over previous
import functools

import jax
import jax.numpy as jnp
from jax import lax
from jax.experimental import pallas as pl
from jax.experimental.pallas import tpu as pltpu
from jax.experimental.pallas import tpu_sc as plsc

NC = 2
NS = 16
NW = NC * NS
CHUNK = 128
K = 4


@functools.partial(jax.jit, static_argnames=("n_chunks", "d"))
def _lookup(emb_table, idx3, *, n_chunks, d):
    b_total = NW * n_chunks * CHUNK
    assert n_chunks % (2 * K) == 0
    n_groups = n_chunks // K
    rows_per_w = n_chunks * CHUNK
    gchunk = K * CHUNK
    mesh = plsc.VectorSubcoreMesh(core_axis_name="c", subcore_axis_name="s")

    def body(table_hbm, idx_hbm, out_hbm, idx_v, rows_v, gsem, wsem):
        wid = lax.axis_index("s") * NC + lax.axis_index("c")
        pltpu.sync_copy(idx_hbm.at[wid], idx_v)
        base = wid * rows_per_w

        def gathers(g, b):
            for j in range(K):
                pltpu.async_copy(
                    table_hbm.at[idx_v.at[g * K + j]],
                    rows_v.at[b, pl.ds(j * CHUNK, CHUNK)],
                    gsem.at[b],
                )

        def drain_gathers(b):
            pltpu.make_async_copy(
                table_hbm.at[pl.ds(0, gchunk)], rows_v.at[b], gsem.at[b]
            ).wait()

        def write_group(g, b):
            return pltpu.make_async_copy(
                rows_v.at[b],
                out_hbm.at[pl.ds(base + g * gchunk, gchunk)],
                wsem.at[b],
            )

        gathers(0, 0)

        def outer(s, carry):
            for b in range(2):
                g = s * 2 + b
                nb = 1 - b

                @pl.when(g + 1 < n_groups)
                def _():
                    @pl.when(g >= 1)
                    def _():
                        write_group(g - 1, nb).wait()
                    gathers(g + 1, nb)

                drain_gathers(b)
                write_group(g, b).start()
            return carry

        lax.fori_loop(0, n_groups // 2, outer, 0)
        write_group(n_groups - 2, (n_groups - 2) % 2).wait()
        write_group(n_groups - 1, (n_groups - 1) % 2).wait()

    k = pl.kernel(
        body,
        out_type=jax.ShapeDtypeStruct((b_total, d), jnp.float32),
        mesh=mesh,
        compiler_params=pltpu.CompilerParams(use_tc_tiling_on_sc=False),
        scratch_types=[
            pltpu.VMEM((n_chunks, CHUNK), jnp.int32),
            pltpu.VMEM((2, gchunk, d), jnp.float32),
            pltpu.SemaphoreType.DMA((2,)),
            pltpu.SemaphoreType.DMA((2,)),
        ],
    )
    return k(emb_table, idx3)


def kernel(x, emb_table):
    b_total = x.size
    d = emb_table.shape[1]
    assert b_total % (NW * CHUNK) == 0
    n_chunks = b_total // (NW * CHUNK)
    idx3 = x.reshape(NW, n_chunks, CHUNK).astype(jnp.int32)
    out = _lookup(emb_table, idx3, n_chunks=n_chunks, d=d)
    return out.reshape(x.shape + (d,))
